# baseline (speedup 1.0000x reference)
import functools

import jax
import jax.numpy as jnp
from jax import lax
from jax.experimental import pallas as pl
from jax.experimental.pallas import tpu as pltpu
from jax.experimental.pallas import tpu_sc as plsc

N = 10000
R = 20
H = 3
E = 320000
TN = 1000
NW = 32
EPW = E // NW
C = 80
NCH = EPW // C
NPS = 624
TAIL = N - 16 * NPS

_SC_PARAMS = pltpu.CompilerParams(use_tc_tiling_on_sc=False)
_MESH = dict(core_axis_name="c", subcore_axis_name="s",
             num_cores=2, num_subcores=16)


def _trans_body(h_ref, w_ref, alt_ref, art_ref, t2_ref, lg_ref):
    t = jnp.dot(h_ref[...], w_ref[0], preferred_element_type=jnp.float32)
    hw = t.shape[1] // 2
    t2_ref[0, 0] = t[:, :hw]
    t2_ref[1, 0] = t[:, hw:]
    el = jnp.dot(t, alt_ref[...], preferred_element_type=jnp.float32)
    er = jnp.dot(t, art_ref[...], preferred_element_type=jnp.float32)
    lg_ref[0] = jnp.concatenate([el, er], axis=1)


def _trans_call(h, W_rel, a_l, a_r):
    in_dim = h.shape[1]
    ho = W_rel.shape[2]
    hw = ho // 2
    out = ho // H
    heads = jnp.arange(ho) // out
    offs = jnp.arange(ho) % out
    cols = jnp.arange(8)[None, :]
    alt = jnp.where(cols == heads[:, None], a_l[heads, offs][:, None], 0.0)
    art = jnp.where(cols == heads[:, None], a_r[heads, offs][:, None], 0.0)
    grid = (R, N // TN)
    return pl.pallas_call(
        _trans_body,
        grid=grid,
        in_specs=[
            pl.BlockSpec((TN, in_dim), lambda r, t: (t, 0)),
            pl.BlockSpec((1, in_dim, ho), lambda r, t: (r, 0, 0)),
            pl.BlockSpec((ho, 8), lambda r, t: (0, 0)),
            pl.BlockSpec((ho, 8), lambda r, t: (0, 0)),
        ],
        out_specs=[
            pl.BlockSpec((2, 1, TN, hw), lambda r, t: (0, r, t, 0)),
            pl.BlockSpec((1, TN, 16), lambda r, t: (r, t, 0)),
        ],
        out_shape=[
            jax.ShapeDtypeStruct((2, R, N, hw), jnp.float32),
            jax.ShapeDtypeStruct((R, N, 16), jnp.float32),
        ],
    )(h, W_rel, alt, art)


def _vgather16(v, idx):
    dn = lax.GatherDimensionNumbers(
        offset_dims=(), collapsed_slice_dims=(0,), start_index_map=(0,))
    return lax.gather(v, idx[:, None], dn, (1,),
                      mode=lax.GatherScatterMode.PROMISE_IN_BOUNDS)


def _zero_shared(zero_h, sh, ss):
    pltpu.sync_copy(zero_h.at[pl.ds(ss * NPS, NPS)], sh.at[pl.ds(ss * NPS, NPS)])
    @pl.when(ss == 15)
    def _tail():
        pltpu.sync_copy(zero_h.at[pl.ds(16 * NPS, TAIL)],
                        sh.at[pl.ds(16 * NPS, TAIL)])


def _writeout_shared(sh, out_h, cc, ss):
    pltpu.sync_copy(sh.at[pl.ds(ss * NPS, NPS)],
                    out_h.at[cc, pl.ds(ss * NPS, NPS)])
    @pl.when(ss == 15)
    def _tail():
        pltpu.sync_copy(sh.at[pl.ds(16 * NPS, TAIL)],
                        out_h.at[cc, pl.ds(16 * NPS, TAIL)])


def _edge_pass_call(lg, trans2, rn_src, rn_dst, dst):
    hw = trans2.shape[2]
    ng = hw // 16
    gph = (2 * hw) // (16 * H)
    W = hw + 16
    EPS = E // 16
    NCH2 = EPS // C
    zeros = jnp.zeros((N, W), jnp.float32)
    mesh = plsc.VectorSubcoreMesh(**_MESH)

    @functools.partial(
        pl.kernel,
        out_type=jax.ShapeDtypeStruct((2, N, W), jnp.float32),
        mesh=mesh,
        compiler_params=_SC_PARAMS,
        scratch_types=[
            pltpu.VMEM_SHARED((N, W), jnp.float32),
            pltpu.VMEM((3, C), jnp.int32),
            pltpu.VMEM((3, C), jnp.int32),
            pltpu.VMEM((3, C), jnp.int32),
            pltpu.VMEM((2, C, 16), jnp.float32),
            pltpu.VMEM((2, C, 16), jnp.float32),
            pltpu.VMEM((2, C, hw), jnp.float32),
            pltpu.VMEM((C, W), jnp.float32),
            pltpu.SemaphoreType.DMA((3,)),
            pltpu.SemaphoreType.DMA((2,)),
        ],
    )
    def k(lg_h, t2_h, rns_h, rnd_h, dst_h, zero_h, agg_out,
          acc_sh, rns_v, rnd_v, dst_v, lgs_v, lgd_v, msg_v, sc_v, semL, semG):
        cc = lax.axis_index("c")
        ss = lax.axis_index("s")
        base0 = ss * EPS
        lane = lax.iota(jnp.int32, 16)
        shift_idx = jnp.where(lane < H, lane + 8, 0)
        is_lo = cc == 0
        _zero_shared(zero_h, acc_sh, ss)
        plsc.subcore_barrier()

        def fire_L(kk):
            s3 = kk % 3
            base = base0 + kk * C
            pltpu.async_copy(rns_h.at[pl.ds(base, C)], rns_v.at[s3], semL.at[s3])
            pltpu.async_copy(rnd_h.at[pl.ds(base, C)], rnd_v.at[s3], semL.at[s3])
            pltpu.async_copy(dst_h.at[pl.ds(base, C)], dst_v.at[s3], semL.at[s3])

        def wait_L(kk):
            s3 = kk % 3
            pltpu.make_async_copy(rns_h.at[pl.ds(0, C)], rns_v.at[s3], semL.at[s3]).wait()
            pltpu.make_async_copy(rnd_h.at[pl.ds(0, C)], rnd_v.at[s3], semL.at[s3]).wait()
            pltpu.make_async_copy(dst_h.at[pl.ds(0, C)], dst_v.at[s3], semL.at[s3]).wait()

        def fire_G(kk):
            s3 = kk % 3
            s2 = kk % 2
            pltpu.async_copy(lg_h.at[rns_v.at[s3]], lgs_v.at[s2], semG.at[s2])
            pltpu.async_copy(lg_h.at[rnd_v.at[s3]], lgd_v.at[s2], semG.at[s2])
            pltpu.async_copy(t2_h.at[cc].at[rns_v.at[s3]], msg_v.at[s2], semG.at[s2])

        def wait_G(kk):
            s3 = kk % 3
            s2 = kk % 2
            pltpu.make_async_copy(lg_h.at[rns_v.at[s3]], lgs_v.at[s2], semG.at[s2]).wait()
            pltpu.make_async_copy(lg_h.at[rnd_v.at[s3]], lgd_v.at[s2], semG.at[s2]).wait()
            pltpu.make_async_copy(t2_h.at[cc].at[rns_v.at[s3]], msg_v.at[s2], semG.at[s2]).wait()

        def compute(kk):
            s2 = kk % 2

            def edge_body(i, carry):
                e = lgs_v[s2, i] + _vgather16(lgd_v[s2, i], shift_idx)
                e = jnp.where(e >= 0.0, e, 0.2 * e)
                ex = jnp.where(lane < H, jnp.exp(e), 0.0)
                sc_v[i, pl.ds(hw, 16)] = jnp.where(is_lo, ex, 0.0)
                for g in range(ng):
                    hsel = jnp.where(is_lo, g // gph, (ng + g) // gph)
                    b = _vgather16(ex, lane * 0 + hsel)
                    sc_v[i, pl.ds(g * 16, 16)] = (
                        msg_v[s2, i, pl.ds(g * 16, 16)] * b)
                return carry

            lax.fori_loop(0, C, edge_body, 0, unroll=4)

        fire_L(0)
        wait_L(0)
        fire_G(0)
        fire_L(1)

        def chunk_body(kk, carry):
            wait_G(kk)
            @pl.when(kk + 1 < NCH2)
            def _next_g():
                wait_L(kk + 1)
                fire_G(kk + 1)
            compute(kk)
            pltpu.sync_copy(sc_v, acc_sh.at[dst_v.at[kk % 3]], add=True)
            @pl.when(kk + 2 < NCH2)
            def _next_l():
                fire_L(kk + 2)
            return carry

        lax.fori_loop(0, NCH2, chunk_body, 0)
        plsc.subcore_barrier()
        _writeout_shared(acc_sh, agg_out, cc, ss)

    return k(lg, trans2, rn_src, rn_dst, dst, zeros)


def _combine_body(h_ref, ws_ref, agg_ref, exp_ref, o_ref, *, relu, hw):
    s = jnp.dot(h_ref[...], ws_ref[...], preferred_element_type=jnp.float32)
    lo = agg_ref[0]
    hi = agg_ref[1]
    a = jnp.concatenate([lo[:, :hw], hi[:, :hw]], axis=1)
    d = lo[:, hw:]
    denf = jnp.dot(d, exp_ref[...], preferred_element_type=jnp.float32)
    o = a / (denf + 1e-9) + s
    o_ref[...] = jnp.maximum(o, 0.0) if relu else o


def _combine_call(h, W_self, agg2, relu):
    in_dim = h.shape[1]
    ho = W_self.shape[1]
    hw = ho // 2
    out = ho // H
    expand = (jnp.arange(16)[:, None] == (jnp.arange(ho) // out)[None, :]).astype(jnp.float32)
    grid = (N // TN,)
    return pl.pallas_call(
        functools.partial(_combine_body, relu=relu, hw=hw),
        grid=grid,
        in_specs=[
            pl.BlockSpec((TN, in_dim), lambda t: (t, 0)),
            pl.BlockSpec((in_dim, ho), lambda t: (0, 0)),
            pl.BlockSpec((2, TN, hw + 16), lambda t: (0, t, 0)),
            pl.BlockSpec((16, ho), lambda t: (0, 0)),
        ],
        out_specs=pl.BlockSpec((TN, ho), lambda t: (t, 0)),
        out_shape=jax.ShapeDtypeStruct((N, ho), jnp.float32),
    )(h, W_self, agg2, expand)


def kernel(features, edge_index, edge_type, W_rel_0, a_l_0, a_r_0, W_self_0,
           W_rel_1, a_l_1, a_r_1, W_self_1, W_rel_2, a_l_2, a_r_2, W_self_2):
    src = edge_index[0]
    dst = edge_index[1]
    rn_src = edge_type * N + src
    rn_dst = edge_type * N + dst
    h = features
    layers = [
        (W_rel_0, a_l_0, a_r_0, W_self_0, True),
        (W_rel_1, a_l_1, a_r_1, W_self_1, True),
        (W_rel_2, a_l_2, a_r_2, W_self_2, False),
    ]
    for W_rel, a_l, a_r, W_self, relu in layers:
        ho = W_rel.shape[2]
        hw = ho // 2
        t2, lg = _trans_call(h, W_rel, a_l, a_r)
        agg2 = _edge_pass_call(lg.reshape(R * N, 16),
                               t2.reshape(2, R * N, hw), rn_src, rn_dst, dst)
        h = _combine_call(h, W_self, agg2, relu)
    return h


# speedup vs baseline: 1.0167x; 1.0167x over previous
import functools

import jax
import jax.numpy as jnp
from jax import lax
from jax.experimental import pallas as pl
from jax.experimental.pallas import tpu as pltpu
from jax.experimental.pallas import tpu_sc as plsc

N = 10000
R = 20
H = 3
E = 320000
TN = 1000
NW = 32
EPW = E // NW
C = 80
NCH = EPW // C
NPS = 624
TAIL = N - 16 * NPS

_SC_PARAMS = pltpu.CompilerParams(use_tc_tiling_on_sc=False)
_MESH = dict(core_axis_name="c", subcore_axis_name="s",
             num_cores=2, num_subcores=16)


def _trans_body(h_ref, w_ref, alt_ref, art_ref, t2_ref, lg_ref):
    t = jnp.dot(h_ref[...], w_ref[0], preferred_element_type=jnp.float32)
    hw = t.shape[1] // 2
    t2_ref[0, 0] = t[:, :hw]
    t2_ref[1, 0] = t[:, hw:]
    el = jnp.dot(t, alt_ref[...], preferred_element_type=jnp.float32)
    er = jnp.dot(t, art_ref[...], preferred_element_type=jnp.float32)
    lg_ref[0] = jnp.concatenate([el, er], axis=1)


def _trans_call(h, W_rel, a_l, a_r):
    in_dim = h.shape[1]
    ho = W_rel.shape[2]
    hw = ho // 2
    out = ho // H
    heads = jnp.arange(ho) // out
    offs = jnp.arange(ho) % out
    cols = jnp.arange(8)[None, :]
    alt = jnp.where(cols == heads[:, None], a_l[heads, offs][:, None], 0.0)
    art = jnp.where(cols == heads[:, None], a_r[heads, offs][:, None], 0.0)
    grid = (R, N // TN)
    return pl.pallas_call(
        _trans_body,
        grid=grid,
        in_specs=[
            pl.BlockSpec((TN, in_dim), lambda r, t: (t, 0)),
            pl.BlockSpec((1, in_dim, ho), lambda r, t: (r, 0, 0)),
            pl.BlockSpec((ho, 8), lambda r, t: (0, 0)),
            pl.BlockSpec((ho, 8), lambda r, t: (0, 0)),
        ],
        out_specs=[
            pl.BlockSpec((2, 1, TN, hw), lambda r, t: (0, r, t, 0)),
            pl.BlockSpec((1, TN, 16), lambda r, t: (r, t, 0)),
        ],
        out_shape=[
            jax.ShapeDtypeStruct((2, R, N, hw), jnp.float32),
            jax.ShapeDtypeStruct((R, N, 16), jnp.float32),
        ],
    )(h, W_rel, alt, art)


def _vgather16(v, idx):
    dn = lax.GatherDimensionNumbers(
        offset_dims=(), collapsed_slice_dims=(0,), start_index_map=(0,))
    return lax.gather(v, idx[:, None], dn, (1,),
                      mode=lax.GatherScatterMode.PROMISE_IN_BOUNDS)


def _zero_shared(zero_h, sh, ss):
    pltpu.sync_copy(zero_h.at[pl.ds(ss * NPS, NPS)], sh.at[pl.ds(ss * NPS, NPS)])
    @pl.when(ss == 15)
    def _tail():
        pltpu.sync_copy(zero_h.at[pl.ds(16 * NPS, TAIL)],
                        sh.at[pl.ds(16 * NPS, TAIL)])


def _writeout_shared(sh, out_h, cc, ss):
    pltpu.sync_copy(sh.at[pl.ds(ss * NPS, NPS)],
                    out_h.at[cc, pl.ds(ss * NPS, NPS)])
    @pl.when(ss == 15)
    def _tail():
        pltpu.sync_copy(sh.at[pl.ds(16 * NPS, TAIL)],
                        out_h.at[cc, pl.ds(16 * NPS, TAIL)])


def _edge_pass_call(lg, trans2, rn_src, rn_dst, dst, half):
    hw = trans2.shape[2]
    ng = hw // 16
    gph = (2 * hw) // (16 * H)
    with_ex = half == 0
    W = hw + 16 if with_ex else hw
    zeros = jnp.zeros((N, W), jnp.float32)
    mesh = plsc.VectorSubcoreMesh(**_MESH)

    @functools.partial(
        pl.kernel,
        out_type=jax.ShapeDtypeStruct((2, N, W), jnp.float32),
        mesh=mesh,
        compiler_params=_SC_PARAMS,
        scratch_types=[
            pltpu.VMEM_SHARED((N, W), jnp.float32),
            pltpu.VMEM((3, C), jnp.int32),
            pltpu.VMEM((3, C), jnp.int32),
            pltpu.VMEM((3, C), jnp.int32),
            pltpu.VMEM((2, C, 16), jnp.float32),
            pltpu.VMEM((2, C, 16), jnp.float32),
            pltpu.VMEM((2, C, hw), jnp.float32),
            pltpu.VMEM((C, W), jnp.float32),
            pltpu.SemaphoreType.DMA((3,)),
            pltpu.SemaphoreType.DMA((2,)),
        ],
    )
    def k(lg_h, t2_h, rns_h, rnd_h, dst_h, zero_h, agg_out,
          acc_sh, rns_v, rnd_v, dst_v, lgs_v, lgd_v, msg_v, sc_v, semL, semG):
        cc = lax.axis_index("c")
        ss = lax.axis_index("s")
        base0 = (cc * 16 + ss) * EPW
        lane = lax.iota(jnp.int32, 16)
        shift_idx = jnp.where(lane < H, lane + 8, 0)
        _zero_shared(zero_h, acc_sh, ss)
        plsc.subcore_barrier()

        def fire_L(kk):
            s3 = kk % 3
            base = base0 + kk * C
            pltpu.async_copy(rns_h.at[pl.ds(base, C)], rns_v.at[s3], semL.at[s3])
            pltpu.async_copy(rnd_h.at[pl.ds(base, C)], rnd_v.at[s3], semL.at[s3])
            pltpu.async_copy(dst_h.at[pl.ds(base, C)], dst_v.at[s3], semL.at[s3])

        def wait_L(kk):
            s3 = kk % 3
            pltpu.make_async_copy(rns_h.at[pl.ds(0, C)], rns_v.at[s3], semL.at[s3]).wait()
            pltpu.make_async_copy(rnd_h.at[pl.ds(0, C)], rnd_v.at[s3], semL.at[s3]).wait()
            pltpu.make_async_copy(dst_h.at[pl.ds(0, C)], dst_v.at[s3], semL.at[s3]).wait()

        def fire_G(kk):
            s3 = kk % 3
            s2 = kk % 2
            pltpu.async_copy(lg_h.at[rns_v.at[s3]], lgs_v.at[s2], semG.at[s2])
            pltpu.async_copy(lg_h.at[rnd_v.at[s3]], lgd_v.at[s2], semG.at[s2])
            pltpu.async_copy(t2_h.at[half].at[rns_v.at[s3]], msg_v.at[s2], semG.at[s2])

        def wait_G(kk):
            s3 = kk % 3
            s2 = kk % 2
            pltpu.make_async_copy(lg_h.at[rns_v.at[s3]], lgs_v.at[s2], semG.at[s2]).wait()
            pltpu.make_async_copy(lg_h.at[rnd_v.at[s3]], lgd_v.at[s2], semG.at[s2]).wait()
            pltpu.make_async_copy(t2_h.at[half].at[rns_v.at[s3]], msg_v.at[s2], semG.at[s2]).wait()

        def compute(kk):
            s2 = kk % 2

            def edge_body(i, carry):
                e = lgs_v[s2, i] + _vgather16(lgd_v[s2, i], shift_idx)
                e = jnp.where(e >= 0.0, e, 0.2 * e)
                ex = jnp.where(lane < H, jnp.exp(e), 0.0)
                if with_ex:
                    sc_v[i, pl.ds(hw, 16)] = ex
                b = [_vgather16(ex, lane * 0 + hh) for hh in range(H)]
                for g in range(ng):
                    hh = (half * ng + g) // gph
                    sc_v[i, pl.ds(g * 16, 16)] = (
                        msg_v[s2, i, pl.ds(g * 16, 16)] * b[hh])
                return carry

            lax.fori_loop(0, C, edge_body, 0, unroll=4)

        fire_L(0)
        wait_L(0)
        fire_G(0)
        fire_L(1)

        def chunk_body(kk, carry):
            wait_G(kk)
            @pl.when(kk + 1 < NCH)
            def _next_g():
                wait_L(kk + 1)
                fire_G(kk + 1)
            compute(kk)
            pltpu.sync_copy(sc_v, acc_sh.at[dst_v.at[kk % 3]], add=True)
            @pl.when(kk + 2 < NCH)
            def _next_l():
                fire_L(kk + 2)
            return carry

        lax.fori_loop(0, NCH, chunk_body, 0)
        plsc.subcore_barrier()
        _writeout_shared(acc_sh, agg_out, cc, ss)

    return k(lg, trans2, rn_src, rn_dst, dst, zeros)


def _combine_body(h_ref, ws_ref, alo_ref, ahi_ref, exp_ref, o_ref, *, relu, hw):
    s = jnp.dot(h_ref[...], ws_ref[...], preferred_element_type=jnp.float32)
    lo = alo_ref[0] + alo_ref[1]
    hi = ahi_ref[0] + ahi_ref[1]
    a = jnp.concatenate([lo[:, :hw], hi], axis=1)
    d = lo[:, hw:]
    denf = jnp.dot(d, exp_ref[...], preferred_element_type=jnp.float32)
    o = a / (denf + 1e-9) + s
    o_ref[...] = jnp.maximum(o, 0.0) if relu else o


def _combine_call(h, W_self, agg_lo, agg_hi, relu):
    in_dim = h.shape[1]
    ho = W_self.shape[1]
    hw = ho // 2
    out = ho // H
    expand = (jnp.arange(16)[:, None] == (jnp.arange(ho) // out)[None, :]).astype(jnp.float32)
    grid = (N // TN,)
    return pl.pallas_call(
        functools.partial(_combine_body, relu=relu, hw=hw),
        grid=grid,
        in_specs=[
            pl.BlockSpec((TN, in_dim), lambda t: (t, 0)),
            pl.BlockSpec((in_dim, ho), lambda t: (0, 0)),
            pl.BlockSpec((2, TN, hw + 16), lambda t: (0, t, 0)),
            pl.BlockSpec((2, TN, hw), lambda t: (0, t, 0)),
            pl.BlockSpec((16, ho), lambda t: (0, 0)),
        ],
        out_specs=pl.BlockSpec((TN, ho), lambda t: (t, 0)),
        out_shape=jax.ShapeDtypeStruct((N, ho), jnp.float32),
    )(h, W_self, agg_lo, agg_hi, expand)


def kernel(features, edge_index, edge_type, W_rel_0, a_l_0, a_r_0, W_self_0,
           W_rel_1, a_l_1, a_r_1, W_self_1, W_rel_2, a_l_2, a_r_2, W_self_2):
    src = edge_index[0]
    dst = edge_index[1]
    rn_src = edge_type * N + src
    rn_dst = edge_type * N + dst
    h = features
    layers = [
        (W_rel_0, a_l_0, a_r_0, W_self_0, True),
        (W_rel_1, a_l_1, a_r_1, W_self_1, True),
        (W_rel_2, a_l_2, a_r_2, W_self_2, False),
    ]
    for W_rel, a_l, a_r, W_self, relu in layers:
        ho = W_rel.shape[2]
        hw = ho // 2
        t2, lg = _trans_call(h, W_rel, a_l, a_r)
        lgf = lg.reshape(R * N, 16)
        t2f = t2.reshape(2, R * N, hw)
        agg_lo = _edge_pass_call(lgf, t2f, rn_src, rn_dst, dst, 0)
        agg_hi = _edge_pass_call(lgf, t2f, rn_src, rn_dst, dst, 1)
        h = _combine_call(h, W_self, agg_lo, agg_hi, relu)
    return h


# speedup vs baseline: 1.0751x; 1.0574x over previous
import functools

import jax
import jax.numpy as jnp
from jax import lax
from jax.experimental import pallas as pl
from jax.experimental.pallas import tpu as pltpu
from jax.experimental.pallas import tpu_sc as plsc

N = 10000
R = 20
H = 3
E = 320000
TN = 1000
NW = 32
EPW = E // NW
C = 80
NCH = EPW // C
NPS = 624
TAIL = N - 16 * NPS

_SC_PARAMS = pltpu.CompilerParams(use_tc_tiling_on_sc=False)
_MESH = dict(core_axis_name="c", subcore_axis_name="s",
             num_cores=2, num_subcores=16)


def _trans_body(h_ref, w_ref, alt_ref, art_ref, tlo_ref, thi_ref, lg_ref):
    t = jnp.dot(h_ref[...], w_ref[0], preferred_element_type=jnp.float32)
    hw = t.shape[1] // 2
    tlo_ref[0] = t[:, :hw]
    thi_ref[0] = t[:, hw:]
    el = jnp.dot(t, alt_ref[...], preferred_element_type=jnp.float32)
    er = jnp.dot(t, art_ref[...], preferred_element_type=jnp.float32)
    lg_ref[0] = jnp.concatenate([el, er], axis=1)


def _trans_call(h, W_rel, a_l, a_r):
    in_dim = h.shape[1]
    ho = W_rel.shape[2]
    hw = ho // 2
    out = ho // H
    heads = jnp.arange(ho) // out
    offs = jnp.arange(ho) % out
    cols = jnp.arange(8)[None, :]
    alt = jnp.where(cols == heads[:, None], a_l[heads, offs][:, None], 0.0)
    art = jnp.where(cols == heads[:, None], a_r[heads, offs][:, None], 0.0)
    grid = (R, N // TN)
    return pl.pallas_call(
        _trans_body,
        grid=grid,
        in_specs=[
            pl.BlockSpec((TN, in_dim), lambda r, t: (t, 0)),
            pl.BlockSpec((1, in_dim, ho), lambda r, t: (r, 0, 0)),
            pl.BlockSpec((ho, 8), lambda r, t: (0, 0)),
            pl.BlockSpec((ho, 8), lambda r, t: (0, 0)),
        ],
        out_specs=[
            pl.BlockSpec((1, TN, hw), lambda r, t: (r, t, 0)),
            pl.BlockSpec((1, TN, hw), lambda r, t: (r, t, 0)),
            pl.BlockSpec((1, TN, 16), lambda r, t: (r, t, 0)),
        ],
        out_shape=[
            jax.ShapeDtypeStruct((R, N, hw), jnp.float32),
            jax.ShapeDtypeStruct((R, N, hw), jnp.float32),
            jax.ShapeDtypeStruct((R, N, 16), jnp.float32),
        ],
    )(h, W_rel, alt, art)


def _vgather16(v, idx):
    dn = lax.GatherDimensionNumbers(
        offset_dims=(), collapsed_slice_dims=(0,), start_index_map=(0,))
    return lax.gather(v, idx[:, None], dn, (1,),
                      mode=lax.GatherScatterMode.PROMISE_IN_BOUNDS)


def _zero_shared(zero_h, sh, ss):
    pltpu.sync_copy(zero_h.at[pl.ds(ss * NPS, NPS)], sh.at[pl.ds(ss * NPS, NPS)])
    @pl.when(ss == 15)
    def _tail():
        pltpu.sync_copy(zero_h.at[pl.ds(16 * NPS, TAIL)],
                        sh.at[pl.ds(16 * NPS, TAIL)])


def _writeout_shared(sh, out_h, cc, ss):
    pltpu.sync_copy(sh.at[pl.ds(ss * NPS, NPS)],
                    out_h.at[cc, pl.ds(ss * NPS, NPS)])
    @pl.when(ss == 15)
    def _tail():
        pltpu.sync_copy(sh.at[pl.ds(16 * NPS, TAIL)],
                        out_h.at[cc, pl.ds(16 * NPS, TAIL)])


def _edge_pass_call(lg, trans_half, rn_src, rn_dst, dst, half):
    hw = trans_half.shape[1]
    ng = hw // 16
    gph = (2 * hw) // (16 * H)
    with_ex = half == 0
    W = hw + 16 if with_ex else hw
    zeros = jnp.zeros((N, W), jnp.float32)
    mesh = plsc.VectorSubcoreMesh(**_MESH)

    @functools.partial(
        pl.kernel,
        out_type=jax.ShapeDtypeStruct((2, N, W), jnp.float32),
        mesh=mesh,
        compiler_params=_SC_PARAMS,
        scratch_types=[
            pltpu.VMEM_SHARED((N, W), jnp.float32),
            pltpu.VMEM((3, C), jnp.int32),
            pltpu.VMEM((3, C), jnp.int32),
            pltpu.VMEM((3, C), jnp.int32),
            pltpu.VMEM((2, C, 16), jnp.float32),
            pltpu.VMEM((2, C, 16), jnp.float32),
            pltpu.VMEM((2, C, hw), jnp.float32),
            pltpu.VMEM((C, W), jnp.float32),
            pltpu.SemaphoreType.DMA((3,)),
            pltpu.SemaphoreType.DMA((2,)),
        ],
    )
    def k(lg_h, th_h, rns_h, rnd_h, dst_h, zero_h, agg_out,
          acc_sh, rns_v, rnd_v, dst_v, lgs_v, lgd_v, msg_v, sc_v, semL, semG):
        cc = lax.axis_index("c")
        ss = lax.axis_index("s")
        base0 = (cc * 16 + ss) * EPW
        lane = lax.iota(jnp.int32, 16)
        shift_idx = jnp.where(lane < H, lane + 8, 0)
        _zero_shared(zero_h, acc_sh, ss)
        plsc.subcore_barrier()

        def fire_L(kk):
            s3 = kk % 3
            base = base0 + kk * C
            pltpu.async_copy(rns_h.at[pl.ds(base, C)], rns_v.at[s3], semL.at[s3])
            pltpu.async_copy(rnd_h.at[pl.ds(base, C)], rnd_v.at[s3], semL.at[s3])
            pltpu.async_copy(dst_h.at[pl.ds(base, C)], dst_v.at[s3], semL.at[s3])

        def wait_L(kk):
            s3 = kk % 3
            pltpu.make_async_copy(rns_h.at[pl.ds(0, C)], rns_v.at[s3], semL.at[s3]).wait()
            pltpu.make_async_copy(rnd_h.at[pl.ds(0, C)], rnd_v.at[s3], semL.at[s3]).wait()
            pltpu.make_async_copy(dst_h.at[pl.ds(0, C)], dst_v.at[s3], semL.at[s3]).wait()

        def fire_G(kk):
            s3 = kk % 3
            s2 = kk % 2
            pltpu.async_copy(lg_h.at[rns_v.at[s3]], lgs_v.at[s2], semG.at[s2])
            pltpu.async_copy(lg_h.at[rnd_v.at[s3]], lgd_v.at[s2], semG.at[s2])
            pltpu.async_copy(th_h.at[rns_v.at[s3]], msg_v.at[s2], semG.at[s2])

        def wait_G(kk):
            s3 = kk % 3
            s2 = kk % 2
            pltpu.make_async_copy(lg_h.at[rns_v.at[s3]], lgs_v.at[s2], semG.at[s2]).wait()
            pltpu.make_async_copy(lg_h.at[rnd_v.at[s3]], lgd_v.at[s2], semG.at[s2]).wait()
            pltpu.make_async_copy(th_h.at[rns_v.at[s3]], msg_v.at[s2], semG.at[s2]).wait()

        def compute(kk):
            s2 = kk % 2

            def edge_body(i, carry):
                e = lgs_v[s2, i] + _vgather16(lgd_v[s2, i], shift_idx)
                e = jnp.where(e >= 0.0, e, 0.2 * e)
                ex = jnp.where(lane < H, jnp.exp(e), 0.0)
                if with_ex:
                    sc_v[i, pl.ds(hw, 16)] = ex
                b = [_vgather16(ex, lane * 0 + hh) for hh in range(H)]
                for g in range(ng):
                    hh = (half * ng + g) // gph
                    sc_v[i, pl.ds(g * 16, 16)] = (
                        msg_v[s2, i, pl.ds(g * 16, 16)] * b[hh])
                return carry

            lax.fori_loop(0, C, edge_body, 0, unroll=4)

        fire_L(0)
        wait_L(0)
        fire_G(0)
        fire_L(1)

        def chunk_body(kk, carry):
            wait_G(kk)
            @pl.when(kk + 1 < NCH)
            def _next_g():
                wait_L(kk + 1)
                fire_G(kk + 1)
            compute(kk)
            pltpu.sync_copy(sc_v, acc_sh.at[dst_v.at[kk % 3]], add=True)
            @pl.when(kk + 2 < NCH)
            def _next_l():
                fire_L(kk + 2)
            return carry

        lax.fori_loop(0, NCH, chunk_body, 0)
        plsc.subcore_barrier()
        _writeout_shared(acc_sh, agg_out, cc, ss)

    return k(lg, trans_half, rn_src, rn_dst, dst, zeros)


def _combine_body(h_ref, ws_ref, alo_ref, ahi_ref, exp_ref, o_ref, *, relu, hw):
    s = jnp.dot(h_ref[...], ws_ref[...], preferred_element_type=jnp.float32)
    lo = alo_ref[0] + alo_ref[1]
    hi = ahi_ref[0] + ahi_ref[1]
    a = jnp.concatenate([lo[:, :hw], hi], axis=1)
    d = lo[:, hw:]
    denf = jnp.dot(d, exp_ref[...], preferred_element_type=jnp.float32)
    o = a / (denf + 1e-9) + s
    o_ref[...] = jnp.maximum(o, 0.0) if relu else o


def _combine_call(h, W_self, agg_lo, agg_hi, relu):
    in_dim = h.shape[1]
    ho = W_self.shape[1]
    hw = ho // 2
    out = ho // H
    expand = (jnp.arange(16)[:, None] == (jnp.arange(ho) // out)[None, :]).astype(jnp.float32)
    grid = (N // TN,)
    return pl.pallas_call(
        functools.partial(_combine_body, relu=relu, hw=hw),
        grid=grid,
        in_specs=[
            pl.BlockSpec((TN, in_dim), lambda t: (t, 0)),
            pl.BlockSpec((in_dim, ho), lambda t: (0, 0)),
            pl.BlockSpec((2, TN, hw + 16), lambda t: (0, t, 0)),
            pl.BlockSpec((2, TN, hw), lambda t: (0, t, 0)),
            pl.BlockSpec((16, ho), lambda t: (0, 0)),
        ],
        out_specs=pl.BlockSpec((TN, ho), lambda t: (t, 0)),
        out_shape=jax.ShapeDtypeStruct((N, ho), jnp.float32),
    )(h, W_self, agg_lo, agg_hi, expand)


def kernel(features, edge_index, edge_type, W_rel_0, a_l_0, a_r_0, W_self_0,
           W_rel_1, a_l_1, a_r_1, W_self_1, W_rel_2, a_l_2, a_r_2, W_self_2):
    src = edge_index[0]
    dst = edge_index[1]
    rn_src = edge_type * N + src
    rn_dst = edge_type * N + dst
    h = features
    layers = [
        (W_rel_0, a_l_0, a_r_0, W_self_0, True),
        (W_rel_1, a_l_1, a_r_1, W_self_1, True),
        (W_rel_2, a_l_2, a_r_2, W_self_2, False),
    ]
    for W_rel, a_l, a_r, W_self, relu in layers:
        ho = W_rel.shape[2]
        hw = ho // 2
        tlo, thi, lg = _trans_call(h, W_rel, a_l, a_r)
        lgf = lg.reshape(R * N, 16)
        agg_lo = _edge_pass_call(lgf, tlo.reshape(R * N, hw), rn_src, rn_dst, dst, 0)
        agg_hi = _edge_pass_call(lgf, thi.reshape(R * N, hw), rn_src, rn_dst, dst, 1)
        h = _combine_call(h, W_self, agg_lo, agg_hi, relu)
    return h
